# baseline (speedup 1.0000x reference)
import functools

import jax
import jax.numpy as jnp
from jax import lax
from jax.experimental import pallas as pl
from jax.experimental.pallas import tpu as pltpu
from jax.experimental.pallas import tpu_sc as plsc

N_DRUGS = 10000
N_PROTS = 10000
D = 256
HD = 128
E = 160000
B = 4096

NC = 2
NS = 16
NW = NC * NS

EPT = E // NS
EK = 80
ENCH = EPT // EK
G = 1
NG = ENCH // G
CR = 80
NCHR = N_DRUGS // CR
RITER = -(-NCHR // NS)


def _dense5_body(hdlo, hdhi, hplo, hphi, wds, wp2d, wd2p, wps, wp2p, bd, bp,
                 obd, obp, omp2d, omd2p, omp2p):
    x_dlo = hdlo[...]
    x_dhi = hdhi[...]
    x_plo = hplo[...]
    x_phi = hphi[...]

    def mm(xlo, xhi, w):
        return (jnp.dot(xlo, w[:HD, :], preferred_element_type=jnp.float32)
                + jnp.dot(xhi, w[HD:, :], preferred_element_type=jnp.float32))

    def store(out_ref, full):
        out_ref[0] = full[:, :HD]
        out_ref[1] = full[:, HD:]

    store(obd, mm(x_dlo, x_dhi, wds[...]) + bd[0, :])
    store(obp, mm(x_plo, x_phi, wps[...]) + bp[0, :])
    store(omp2d, mm(x_plo, x_phi, wp2d[...]))
    store(omd2p, mm(x_dlo, x_dhi, wd2p[...]))
    store(omp2p, mm(x_plo, x_phi, wp2p[...]))


def _dense5(hd_lo, hd_hi, hp_lo, hp_hi, wds, wp2d, wd2p, wps, wp2p, b_d, b_p):
    R = 1000
    grid = (N_DRUGS // R,)
    row_spec = pl.BlockSpec((R, HD), lambda i: (i, 0))
    w_spec = pl.BlockSpec((D, D), lambda i: (0, 0))
    b_spec = pl.BlockSpec((1, D), lambda i: (0, 0))
    out_spec = pl.BlockSpec((2, R, HD), lambda i: (0, i, 0))
    out_sds = jax.ShapeDtypeStruct((2, N_DRUGS, HD), jnp.float32)
    outs = pl.pallas_call(
        _dense5_body,
        grid=grid,
        in_specs=[row_spec, row_spec, row_spec, row_spec,
                  w_spec, w_spec, w_spec, w_spec, w_spec, b_spec, b_spec],
        out_specs=[out_spec] * 5,
        out_shape=[out_sds] * 5,
    )(hd_lo, hd_hi, hp_lo, hp_hi, wds, wp2d, wd2p, wps, wp2p,
      b_d.reshape(1, D), b_p.reshape(1, D))
    return [o.reshape(2 * N_DRUGS, HD) for o in outs]


def _dense3_body(hplo, hphi, wps, wp2d, wp2p, bp, obp, omp2d, omp2p):
    def mm(w):
        return (jnp.dot(hplo[...], w[:HD, :],
                        preferred_element_type=jnp.float32)
                + jnp.dot(hphi[...], w[HD:, :],
                          preferred_element_type=jnp.float32))

    def store(out_ref, full):
        out_ref[0] = full[:, :HD]
        out_ref[1] = full[:, HD:]

    store(obp, mm(wps[...]) + bp[0, :])
    store(omp2d, mm(wp2d[...]))
    store(omp2p, mm(wp2p[...]))


def _dense3(hp_lo, hp_hi, wps, wp2d, wp2p, b_p):
    R = 1000
    grid = (N_PROTS // R,)
    row_spec = pl.BlockSpec((R, HD), lambda i: (i, 0))
    w_spec = pl.BlockSpec((D, D), lambda i: (0, 0))
    b_spec = pl.BlockSpec((1, D), lambda i: (0, 0))
    out_spec = pl.BlockSpec((2, R, HD), lambda i: (0, i, 0))
    out_sds = jax.ShapeDtypeStruct((2, N_PROTS, HD), jnp.float32)
    outs = pl.pallas_call(
        _dense3_body,
        grid=grid,
        in_specs=[row_spec, row_spec, w_spec, w_spec, w_spec, b_spec],
        out_specs=[out_spec] * 3,
        out_shape=[out_sds] * 3,
    )(hp_lo, hp_hi, wps, wp2d, wp2p, b_p.reshape(1, D))
    return [o.reshape(2 * N_PROTS, HD) for o in outs]


def _dense2_body(hdlo, hdhi, hplo, hphi, wds, wp2d, bd, obd, omp2d):
    def mm(xlo, xhi, w):
        return (jnp.dot(xlo[...], w[:HD, :], preferred_element_type=jnp.float32)
                + jnp.dot(xhi[...], w[HD:, :],
                          preferred_element_type=jnp.float32))

    def store(out_ref, full):
        out_ref[0] = full[:, :HD]
        out_ref[1] = full[:, HD:]

    store(obd, mm(hdlo, hdhi, wds[...]) + bd[0, :])
    store(omp2d, mm(hplo, hphi, wp2d[...]))


def _dense2(hd_lo, hd_hi, hp_lo, hp_hi, wds, wp2d, b_d):
    R = 1000
    grid = (N_DRUGS // R,)
    row_spec = pl.BlockSpec((R, HD), lambda i: (i, 0))
    w_spec = pl.BlockSpec((D, D), lambda i: (0, 0))
    b_spec = pl.BlockSpec((1, D), lambda i: (0, 0))
    out_spec = pl.BlockSpec((2, R, HD), lambda i: (0, i, 0))
    out_sds = jax.ShapeDtypeStruct((2, N_DRUGS, HD), jnp.float32)
    outs = pl.pallas_call(
        _dense2_body,
        grid=grid,
        in_specs=[row_spec, row_spec, row_spec, row_spec,
                  w_spec, w_spec, b_spec],
        out_specs=[out_spec] * 2,
        out_shape=[out_sds] * 2,
    )(hd_lo, hd_hi, hp_lo, hp_hi, wds, wp2d, b_d.reshape(1, D))
    return [o.reshape(2 * N_DRUGS, HD) for o in outs]


def _sc_layer_body(phase, *refs):
    if phase == "d":
        (base_d, m_p2d,
         src_dp_d, dst_dp_d,
         out_d,
         acc, idxs_all, idba, idbb, r0b, r1b, gsem, ssem) = refs
    else:
        (base_p, m_d2p, m_p2p,
         src_dp_p, dst_dp_p, src_pp, dst_pp,
         out_p,
         acc, idxs_all, idba, idbb, r0b, r1b, gsem, ssem) = refs
    set_a = ([r0b], idba)
    set_b = ([r1b], idbb)

    c = lax.axis_index("c")
    s = lax.axis_index("s")
    half_row = c * N_DRUGS

    def row_chunks(body):
        @pl.loop(0, RITER)
        def _iter(j):
            cid = s + NS * j
            @pl.when(cid < NCHR)
            def _():
                body(pl.multiple_of(cid * CR, 8))

    def run_phase(base_hbm, out_hbm, ops):
        def init_fire(r0):
            g0 = pl.multiple_of(half_row + r0, 8)
            pltpu.async_copy(base_hbm.at[pl.ds(g0, CR)],
                             acc.at[pl.ds(r0, CR)], gsem)

        def init_drain(r0):
            g0 = pl.multiple_of(half_row + r0, 8)
            pltpu.make_async_copy(base_hbm.at[pl.ds(g0, CR)],
                                  acc.at[pl.ds(r0, CR)], gsem).wait()
        row_chunks(init_fire)
        row_chunks(init_drain)
        plsc.subcore_barrier()

        e_base = pl.multiple_of(s * EPT, 8)
        for (src2, dst1, m) in ops:
            s0 = pl.multiple_of(c * E + e_base, 8)
            pltpu.sync_copy(src2.at[pl.ds(s0, EPT)], idxs_all)

            def g_start(grp, bset):
                bufs, idb = bset
                for b in range(G):
                    ch = grp * G + b
                    d0 = pl.multiple_of(e_base + ch * EK, 8)
                    pltpu.async_copy(dst1.at[pl.ds(d0, EK)], idb.at[b], gsem)
                    isl = idxs_all.at[pl.ds(pl.multiple_of(ch * EK, 8), EK)]
                    pltpu.async_copy(m.at[isl], bufs[b], gsem)

            def g_wait(bset):
                bufs, idb = bset
                for b in range(G):
                    pltpu.make_async_copy(dst1.at[pl.ds(0, EK)], idb.at[b],
                                          gsem).wait()
                    isl = idxs_all.at[pl.ds(0, EK)]
                    pltpu.make_async_copy(m.at[isl], bufs[b], gsem).wait()

            def s_start(grp, bset):
                bufs, idb = bset
                for b in range(G):
                    pltpu.async_copy(bufs[b], acc.at[idb.at[b]],
                                     ssem, add=True)

            def s_wait(bset):
                bufs, idb = bset
                for b in range(G):
                    pltpu.make_async_copy(bufs[b], acc.at[idb.at[0]],
                                          ssem).wait()

            def steady(a):
                g_wait(set_a); s_start(a, set_a)
                g_wait(set_b); s_start(a + 1, set_b)
                s_wait(set_a); g_start(a + 2, set_a)
                s_wait(set_b); g_start(a + 3, set_b)

            g_start(0, set_a)
            g_start(1, set_b)

            @pl.loop(0, (NG - 3) // 2)
            def _grp(kk):
                steady(2 * kk)

            a = NG - 3
            g_wait(set_a); s_start(a, set_a)
            g_wait(set_b); s_start(a + 1, set_b)
            s_wait(set_a); g_start(a + 2, set_a)
            s_wait(set_b)
            g_wait(set_a); s_start(NG - 1, set_a)
            s_wait(set_a)
        plsc.subcore_barrier()

        def epi_fire(r0):
            g0 = pl.multiple_of(half_row + r0, 8)
            pltpu.async_copy(acc.at[pl.ds(r0, CR)],
                             out_hbm.at[pl.ds(g0, CR)], gsem)

        def epi_drain(r0):
            g0 = pl.multiple_of(half_row + r0, 8)
            pltpu.make_async_copy(acc.at[pl.ds(r0, CR)],
                                  out_hbm.at[pl.ds(g0, CR)], gsem).wait()
        row_chunks(epi_fire)
        row_chunks(epi_drain)

    if phase == "d":
        run_phase(base_d, out_d, [(src_dp_d, dst_dp_d, m_p2d)])
    else:
        run_phase(base_p, out_p, [(src_dp_p, dst_dp_p, m_d2p),
                                  (src_pp, dst_pp, m_p2p)])


def _sc_mesh():
    return plsc.VectorSubcoreMesh(core_axis_name="c", subcore_axis_name="s",
                                  num_cores=NC, num_subcores=NS)


def _sc_layer(phase):
    mesh = _sc_mesh()
    out_sds = jax.ShapeDtypeStruct((2 * N_DRUGS, HD), jnp.float32)
    return pl.kernel(
        functools.partial(_sc_layer_body, phase),
        out_type=out_sds,
        mesh=mesh,
        scratch_types=(
            [pltpu.VMEM_SHARED((N_DRUGS, HD), jnp.float32)]
            + [pltpu.VMEM((EPT,), jnp.int32)]
            + [pltpu.VMEM((G, EK), jnp.int32)] * 2
            + [pltpu.VMEM((EK, HD), jnp.float32)] * 2
            + [pltpu.SemaphoreType.DMA, pltpu.SemaphoreType.DMA]
        ),
    )


def _act_relu_body(raw, out):
    out[...] = jnp.maximum(raw[...], 0.0)


def _act_res_body(raw, prev, out):
    out[...] = prev[...] + jnp.maximum(raw[...], 0.0)


def _act(raw, prev=None):
    R = 2000
    grid = (2 * N_DRUGS // R,)
    spec = pl.BlockSpec((R, HD), lambda i: (i, 0))
    if prev is None:
        return pl.pallas_call(
            _act_relu_body, grid=grid, in_specs=[spec], out_specs=spec,
            out_shape=jax.ShapeDtypeStruct((2 * N_DRUGS, HD), jnp.float32),
        )(raw)
    return pl.pallas_call(
        _act_res_body, grid=grid, in_specs=[spec, spec], out_specs=spec,
        out_shape=jax.ShapeDtypeStruct((2 * N_DRUGS, HD), jnp.float32),
    )(raw, prev)


def _pair_gather_body(hd2, idxall, out, idxv, rows, sem):
    wid = lax.axis_index("s") * NC + lax.axis_index("c")
    n = 4 * B // NW
    for j in range(n // 128):
        b0 = wid * n + j * 128
        pltpu.sync_copy(idxall.at[pl.ds(b0, 128)], idxv)
        pltpu.async_copy(hd2.at[idxv], rows, sem).wait()
        pltpu.sync_copy(rows, out.at[pl.ds(b0, 128)])


def _pair_gather(hd2, idx_all):
    mesh = _sc_mesh()
    return pl.kernel(
        _pair_gather_body,
        out_type=jax.ShapeDtypeStruct((4 * B, HD), jnp.float32),
        mesh=mesh,
        scratch_types=[
            pltpu.VMEM((128,), jnp.int32),
            pltpu.VMEM((128, HD), jnp.float32),
            pltpu.SemaphoreType.DMA,
        ],
    )(hd2, idx_all)


def _mlp_body(x0, x1, x2, x3, w1, b1, w2, b2, w3, b3, out):
    h = (jnp.dot(x0[...], w1[0], preferred_element_type=jnp.float32)
         + jnp.dot(x1[...], w1[1], preferred_element_type=jnp.float32)
         + jnp.dot(x2[...], w1[2], preferred_element_type=jnp.float32)
         + jnp.dot(x3[...], w1[3], preferred_element_type=jnp.float32))
    h = jnp.maximum(h + b1[0, :], 0.0)
    h = jnp.maximum(jnp.dot(h, w2[...], preferred_element_type=jnp.float32)
                    + b2[0, :], 0.0)
    out[...] = (jnp.dot(h, w3[...], preferred_element_type=jnp.float32)
                + b3[0, :])


def _mlp(pairs, wp1, bp1, wp2, bp2, wp3, bp3):
    R = 1024
    grid = (B // R,)
    x_spec = pl.BlockSpec((R, HD), lambda i: (i, 0))
    xs = [pairs[k * B:(k + 1) * B] for k in range(4)]
    w3p = jnp.zeros((64, HD), jnp.float32).at[:, :1].set(wp3)
    b3p = jnp.zeros((1, HD), jnp.float32).at[0, 0].set(bp3[0])
    out = pl.pallas_call(
        _mlp_body,
        grid=grid,
        in_specs=[x_spec, x_spec, x_spec, x_spec,
                  pl.BlockSpec((4, HD, HD), lambda i: (0, 0, 0)),
                  pl.BlockSpec((1, HD), lambda i: (0, 0)),
                  pl.BlockSpec((HD, 64), lambda i: (0, 0)),
                  pl.BlockSpec((1, 64), lambda i: (0, 0)),
                  pl.BlockSpec((64, HD), lambda i: (0, 0)),
                  pl.BlockSpec((1, HD), lambda i: (0, 0))],
        out_specs=pl.BlockSpec((R, HD), lambda i: (i, 0)),
        out_shape=jax.ShapeDtypeStruct((B, HD), jnp.float32),
    )(xs[0], xs[1], xs[2], xs[3],
      wp1.reshape(4, HD, HD), bp1.reshape(1, HD),
      wp2, bp2.reshape(1, 64), w3p, b3p)
    return out[:, :1]


def kernel(x_drugs, x_prots, dp_edge_index, pp_edge_index, dd_pair_index,
           prot_emb, W1_d_self, W1_p2d, W1_d2p, W1_p_self, W1_p2p, b1_d, b1_p,
           W_res, b_res, Wp1, bp1, Wp2, bp2, Wp3, bp3):
    i32 = jnp.int32
    dp0 = dp_edge_index[0].astype(i32)
    dp1 = dp_edge_index[1].astype(i32)
    pp0 = pp_edge_index[0].astype(i32)
    pp1 = pp_edge_index[1].astype(i32)

    src_dp_d = jnp.concatenate([dp1, dp1 + N_PROTS])
    src_dp_p = jnp.concatenate([dp0, dp0 + N_DRUGS])
    src_pp = jnp.concatenate([pp0, pp0 + N_PROTS])

    hd_lo, hd_hi = x_drugs[:, :HD], x_drugs[:, HD:]
    hp_lo, hp_hi = prot_emb, x_prots

    bd, md2p = _dense2(hd_lo, hd_hi, hd_lo, hd_hi, W1_d_self, W1_d2p, b1_d)
    bp, mp2d, mp2p = _dense3(hp_lo, hp_hi, W1_p_self, W1_p2d, W1_p2p, b1_p)
    rawd = _sc_layer("d")(bd, mp2d, src_dp_d, dp0)
    rawp = _sc_layer("p")(bp, md2p, mp2p, src_dp_p, dp1, src_pp, pp1)
    hd2, hp2 = _act(rawd), _act(rawp)

    nres = W_res.shape[0]
    for i in range(nres - 1):
        bd, md2p = _dense2(hd2[:N_DRUGS], hd2[N_DRUGS:],
                           hd2[:N_DRUGS], hd2[N_DRUGS:],
                           W_res[i, 0], W_res[i, 2], b_res[i, 0])
        bp, mp2d, mp2p = _dense3(hp2[:N_PROTS], hp2[N_PROTS:],
                                 W_res[i, 3], W_res[i, 1], W_res[i, 4],
                                 b_res[i, 1])
        rawd = _sc_layer("d")(bd, mp2d, src_dp_d, dp0)
        rawp = _sc_layer("p")(bp, md2p, mp2p, src_dp_p, dp1, src_pp, pp1)
        hd2, hp2 = _act(rawd, hd2), _act(rawp, hp2)
    bd, mp2d = _dense2(
        hd2[:N_DRUGS], hd2[N_DRUGS:], hp2[:N_PROTS], hp2[N_PROTS:],
        W_res[nres - 1, 0], W_res[nres - 1, 1], b_res[nres - 1, 0])
    rawd = _sc_layer("d")(bd, mp2d, src_dp_d, dp0)
    hd2 = _act(rawd, hd2)

    pi = dd_pair_index[0].astype(i32)
    pj = dd_pair_index[1].astype(i32)
    idx_all = jnp.concatenate([pi, pi + N_DRUGS, pj, pj + N_DRUGS])
    pairs = _pair_gather(hd2, idx_all)
    comb = _mlp(pairs, Wp1, bp1, Wp2, bp2, Wp3, bp3)
    return comb[:, :, None]


# speedup vs baseline: 1.0187x; 1.0187x over previous
import functools

import jax
import jax.numpy as jnp
from jax import lax
from jax.experimental import pallas as pl
from jax.experimental.pallas import tpu as pltpu
from jax.experimental.pallas import tpu_sc as plsc

N_DRUGS = 10000
N_PROTS = 10000
D = 256
HD = 128
E = 160000
B = 4096

NC = 2
NS = 16
NW = NC * NS

EPT = E // NS
EK = 80
ENCH = EPT // EK
G = 1
NG = ENCH // G
CR = 80
NCHR = N_DRUGS // CR
RITER = -(-NCHR // NS)


def _dense5_body(hdlo, hdhi, hplo, hphi, wds, wp2d, wd2p, wps, wp2p, bd, bp,
                 obd, obp, omp2d, omd2p, omp2p):
    x_dlo = hdlo[...]
    x_dhi = hdhi[...]
    x_plo = hplo[...]
    x_phi = hphi[...]

    def mm(xlo, xhi, w):
        return (jnp.dot(xlo, w[:HD, :], preferred_element_type=jnp.float32)
                + jnp.dot(xhi, w[HD:, :], preferred_element_type=jnp.float32))

    def store(out_ref, full):
        out_ref[0] = full[:, :HD]
        out_ref[1] = full[:, HD:]

    store(obd, mm(x_dlo, x_dhi, wds[...]) + bd[0, :])
    store(obp, mm(x_plo, x_phi, wps[...]) + bp[0, :])
    store(omp2d, mm(x_plo, x_phi, wp2d[...]))
    store(omd2p, mm(x_dlo, x_dhi, wd2p[...]))
    store(omp2p, mm(x_plo, x_phi, wp2p[...]))


def _dense5(hd_lo, hd_hi, hp_lo, hp_hi, wds, wp2d, wd2p, wps, wp2p, b_d, b_p):
    R = 1000
    grid = (N_DRUGS // R,)
    row_spec = pl.BlockSpec((R, HD), lambda i: (i, 0))
    w_spec = pl.BlockSpec((D, D), lambda i: (0, 0))
    b_spec = pl.BlockSpec((1, D), lambda i: (0, 0))
    out_spec = pl.BlockSpec((2, R, HD), lambda i: (0, i, 0))
    out_sds = jax.ShapeDtypeStruct((2, N_DRUGS, HD), jnp.float32)
    outs = pl.pallas_call(
        _dense5_body,
        grid=grid,
        in_specs=[row_spec, row_spec, row_spec, row_spec,
                  w_spec, w_spec, w_spec, w_spec, w_spec, b_spec, b_spec],
        out_specs=[out_spec] * 5,
        out_shape=[out_sds] * 5,
    )(hd_lo, hd_hi, hp_lo, hp_hi, wds, wp2d, wd2p, wps, wp2p,
      b_d.reshape(1, D), b_p.reshape(1, D))
    return [o.reshape(2 * N_DRUGS, HD) for o in outs]


def _dense2_body(hdlo, hdhi, hplo, hphi, wds, wp2d, bd, obd, omp2d):
    def mm(xlo, xhi, w):
        return (jnp.dot(xlo[...], w[:HD, :], preferred_element_type=jnp.float32)
                + jnp.dot(xhi[...], w[HD:, :],
                          preferred_element_type=jnp.float32))

    def store(out_ref, full):
        out_ref[0] = full[:, :HD]
        out_ref[1] = full[:, HD:]

    store(obd, mm(hdlo, hdhi, wds[...]) + bd[0, :])
    store(omp2d, mm(hplo, hphi, wp2d[...]))


def _dense2(hd_lo, hd_hi, hp_lo, hp_hi, wds, wp2d, b_d):
    R = 1000
    grid = (N_DRUGS // R,)
    row_spec = pl.BlockSpec((R, HD), lambda i: (i, 0))
    w_spec = pl.BlockSpec((D, D), lambda i: (0, 0))
    b_spec = pl.BlockSpec((1, D), lambda i: (0, 0))
    out_spec = pl.BlockSpec((2, R, HD), lambda i: (0, i, 0))
    out_sds = jax.ShapeDtypeStruct((2, N_DRUGS, HD), jnp.float32)
    outs = pl.pallas_call(
        _dense2_body,
        grid=grid,
        in_specs=[row_spec, row_spec, row_spec, row_spec,
                  w_spec, w_spec, b_spec],
        out_specs=[out_spec] * 2,
        out_shape=[out_sds] * 2,
    )(hd_lo, hd_hi, hp_lo, hp_hi, wds, wp2d, b_d.reshape(1, D))
    return [o.reshape(2 * N_DRUGS, HD) for o in outs]


def _sc_layer_body(phase, *refs):
    if phase == "d":
        (base_d, m_p2d,
         src_dp_d, dst_dp_d,
         out_d,
         acc, idxs_all, idba, idbb, r0b, r1b, gsem, ssem) = refs
    else:
        (base_p, m_d2p, m_p2p,
         src_dp_p, dst_dp_p, src_pp, dst_pp,
         out_p,
         acc, idxs_all, idba, idbb, r0b, r1b, gsem, ssem) = refs
    set_a = ([r0b], idba)
    set_b = ([r1b], idbb)

    c = lax.axis_index("c")
    s = lax.axis_index("s")
    half_row = c * N_DRUGS

    def row_chunks(body):
        @pl.loop(0, RITER)
        def _iter(j):
            cid = s + NS * j
            @pl.when(cid < NCHR)
            def _():
                body(pl.multiple_of(cid * CR, 8))

    def run_phase(base_hbm, out_hbm, ops):
        def init_fire(r0):
            g0 = pl.multiple_of(half_row + r0, 8)
            pltpu.async_copy(base_hbm.at[pl.ds(g0, CR)],
                             acc.at[pl.ds(r0, CR)], gsem)

        def init_drain(r0):
            g0 = pl.multiple_of(half_row + r0, 8)
            pltpu.make_async_copy(base_hbm.at[pl.ds(g0, CR)],
                                  acc.at[pl.ds(r0, CR)], gsem).wait()
        row_chunks(init_fire)
        row_chunks(init_drain)
        plsc.subcore_barrier()

        e_base = pl.multiple_of(s * EPT, 8)
        for (src2, dst1, m) in ops:
            s0 = pl.multiple_of(c * E + e_base, 8)
            pltpu.sync_copy(src2.at[pl.ds(s0, EPT)], idxs_all)

            def g_start(grp, bset):
                bufs, idb = bset
                for b in range(G):
                    ch = grp * G + b
                    d0 = pl.multiple_of(e_base + ch * EK, 8)
                    pltpu.async_copy(dst1.at[pl.ds(d0, EK)], idb.at[b], gsem)
                    isl = idxs_all.at[pl.ds(pl.multiple_of(ch * EK, 8), EK)]
                    pltpu.async_copy(m.at[isl], bufs[b], gsem)

            def g_wait(bset):
                bufs, idb = bset
                for b in range(G):
                    pltpu.make_async_copy(dst1.at[pl.ds(0, EK)], idb.at[b],
                                          gsem).wait()
                    isl = idxs_all.at[pl.ds(0, EK)]
                    pltpu.make_async_copy(m.at[isl], bufs[b], gsem).wait()

            def s_start(grp, bset):
                bufs, idb = bset
                for b in range(G):
                    pltpu.async_copy(bufs[b], acc.at[idb.at[b]],
                                     ssem, add=True)

            def s_wait(bset):
                bufs, idb = bset
                for b in range(G):
                    pltpu.make_async_copy(bufs[b], acc.at[idb.at[0]],
                                          ssem).wait()

            def steady(a):
                g_wait(set_a); s_start(a, set_a)
                g_wait(set_b); s_start(a + 1, set_b)
                s_wait(set_a); g_start(a + 2, set_a)
                s_wait(set_b); g_start(a + 3, set_b)

            g_start(0, set_a)
            g_start(1, set_b)

            @pl.loop(0, (NG - 3) // 2)
            def _grp(kk):
                steady(2 * kk)

            a = NG - 3
            g_wait(set_a); s_start(a, set_a)
            g_wait(set_b); s_start(a + 1, set_b)
            s_wait(set_a); g_start(a + 2, set_a)
            s_wait(set_b)
            g_wait(set_a); s_start(NG - 1, set_a)
            s_wait(set_a)
        plsc.subcore_barrier()

        def epi_fire(r0):
            g0 = pl.multiple_of(half_row + r0, 8)
            pltpu.async_copy(acc.at[pl.ds(r0, CR)],
                             out_hbm.at[pl.ds(g0, CR)], gsem)

        def epi_drain(r0):
            g0 = pl.multiple_of(half_row + r0, 8)
            pltpu.make_async_copy(acc.at[pl.ds(r0, CR)],
                                  out_hbm.at[pl.ds(g0, CR)], gsem).wait()
        row_chunks(epi_fire)
        row_chunks(epi_drain)

    if phase == "d":
        run_phase(base_d, out_d, [(src_dp_d, dst_dp_d, m_p2d)])
    else:
        run_phase(base_p, out_p, [(src_dp_p, dst_dp_p, m_d2p),
                                  (src_pp, dst_pp, m_p2p)])


def _sc_mesh():
    return plsc.VectorSubcoreMesh(core_axis_name="c", subcore_axis_name="s",
                                  num_cores=NC, num_subcores=NS)


def _sc_layer(phase):
    mesh = _sc_mesh()
    out_sds = jax.ShapeDtypeStruct((2 * N_DRUGS, HD), jnp.float32)
    return pl.kernel(
        functools.partial(_sc_layer_body, phase),
        out_type=out_sds,
        mesh=mesh,
        scratch_types=(
            [pltpu.VMEM_SHARED((N_DRUGS, HD), jnp.float32)]
            + [pltpu.VMEM((EPT,), jnp.int32)]
            + [pltpu.VMEM((G, EK), jnp.int32)] * 2
            + [pltpu.VMEM((EK, HD), jnp.float32)] * 2
            + [pltpu.SemaphoreType.DMA, pltpu.SemaphoreType.DMA]
        ),
    )


def _act_relu_body(raw, out):
    out[...] = jnp.maximum(raw[...], 0.0)


def _act_res_body(raw, prev, out):
    out[...] = prev[...] + jnp.maximum(raw[...], 0.0)


def _act(raw, prev=None):
    R = 2000
    grid = (2 * N_DRUGS // R,)
    spec = pl.BlockSpec((R, HD), lambda i: (i, 0))
    if prev is None:
        return pl.pallas_call(
            _act_relu_body, grid=grid, in_specs=[spec], out_specs=spec,
            out_shape=jax.ShapeDtypeStruct((2 * N_DRUGS, HD), jnp.float32),
        )(raw)
    return pl.pallas_call(
        _act_res_body, grid=grid, in_specs=[spec, spec], out_specs=spec,
        out_shape=jax.ShapeDtypeStruct((2 * N_DRUGS, HD), jnp.float32),
    )(raw, prev)


def _pair_gather_body(hd2, idxall, out, idxv, rows, sem):
    wid = lax.axis_index("s") * NC + lax.axis_index("c")
    n = 4 * B // NW
    for j in range(n // 128):
        b0 = wid * n + j * 128
        pltpu.sync_copy(idxall.at[pl.ds(b0, 128)], idxv)
        pltpu.async_copy(hd2.at[idxv], rows, sem).wait()
        pltpu.sync_copy(rows, out.at[pl.ds(b0, 128)])


def _pair_gather(hd2, idx_all):
    mesh = _sc_mesh()
    return pl.kernel(
        _pair_gather_body,
        out_type=jax.ShapeDtypeStruct((4 * B, HD), jnp.float32),
        mesh=mesh,
        scratch_types=[
            pltpu.VMEM((128,), jnp.int32),
            pltpu.VMEM((128, HD), jnp.float32),
            pltpu.SemaphoreType.DMA,
        ],
    )(hd2, idx_all)


def _mlp_body(x0, x1, x2, x3, w1, b1, w2, b2, w3, b3, out):
    h = (jnp.dot(x0[...], w1[0], preferred_element_type=jnp.float32)
         + jnp.dot(x1[...], w1[1], preferred_element_type=jnp.float32)
         + jnp.dot(x2[...], w1[2], preferred_element_type=jnp.float32)
         + jnp.dot(x3[...], w1[3], preferred_element_type=jnp.float32))
    h = jnp.maximum(h + b1[0, :], 0.0)
    h = jnp.maximum(jnp.dot(h, w2[...], preferred_element_type=jnp.float32)
                    + b2[0, :], 0.0)
    out[...] = (jnp.dot(h, w3[...], preferred_element_type=jnp.float32)
                + b3[0, :])


def _mlp(pairs, wp1, bp1, wp2, bp2, wp3, bp3):
    R = 1024
    grid = (B // R,)
    x_spec = pl.BlockSpec((R, HD), lambda i: (i, 0))
    xs = [pairs[k * B:(k + 1) * B] for k in range(4)]
    w3p = jnp.zeros((64, HD), jnp.float32).at[:, :1].set(wp3)
    b3p = jnp.zeros((1, HD), jnp.float32).at[0, 0].set(bp3[0])
    out = pl.pallas_call(
        _mlp_body,
        grid=grid,
        in_specs=[x_spec, x_spec, x_spec, x_spec,
                  pl.BlockSpec((4, HD, HD), lambda i: (0, 0, 0)),
                  pl.BlockSpec((1, HD), lambda i: (0, 0)),
                  pl.BlockSpec((HD, 64), lambda i: (0, 0)),
                  pl.BlockSpec((1, 64), lambda i: (0, 0)),
                  pl.BlockSpec((64, HD), lambda i: (0, 0)),
                  pl.BlockSpec((1, HD), lambda i: (0, 0))],
        out_specs=pl.BlockSpec((R, HD), lambda i: (i, 0)),
        out_shape=jax.ShapeDtypeStruct((B, HD), jnp.float32),
    )(xs[0], xs[1], xs[2], xs[3],
      wp1.reshape(4, HD, HD), bp1.reshape(1, HD),
      wp2, bp2.reshape(1, 64), w3p, b3p)
    return out[:, :1]


def kernel(x_drugs, x_prots, dp_edge_index, pp_edge_index, dd_pair_index,
           prot_emb, W1_d_self, W1_p2d, W1_d2p, W1_p_self, W1_p2p, b1_d, b1_p,
           W_res, b_res, Wp1, bp1, Wp2, bp2, Wp3, bp3):
    i32 = jnp.int32
    dp0 = dp_edge_index[0].astype(i32)
    dp1 = dp_edge_index[1].astype(i32)
    pp0 = pp_edge_index[0].astype(i32)
    pp1 = pp_edge_index[1].astype(i32)

    src_dp_d = jnp.concatenate([dp1, dp1 + N_PROTS])
    src_dp_p = jnp.concatenate([dp0, dp0 + N_DRUGS])
    src_pp = jnp.concatenate([pp0, pp0 + N_PROTS])

    hd_lo, hd_hi = x_drugs[:, :HD], x_drugs[:, HD:]
    hp_lo, hp_hi = prot_emb, x_prots

    bd, bp, mp2d, md2p, mp2p = _dense5(
        hd_lo, hd_hi, hp_lo, hp_hi,
        W1_d_self, W1_p2d, W1_d2p, W1_p_self, W1_p2p, b1_d, b1_p)
    rawd = _sc_layer("d")(bd, mp2d, src_dp_d, dp0)
    rawp = _sc_layer("p")(bp, md2p, mp2p, src_dp_p, dp1, src_pp, pp1)
    hd2, hp2 = _act(rawd), _act(rawp)

    nres = W_res.shape[0]
    for i in range(nres - 1):
        bd, bp, mp2d, md2p, mp2p = _dense5(
            hd2[:N_DRUGS], hd2[N_DRUGS:], hp2[:N_PROTS], hp2[N_PROTS:],
            W_res[i, 0], W_res[i, 1], W_res[i, 2], W_res[i, 3], W_res[i, 4],
            b_res[i, 0], b_res[i, 1])
        rawd = _sc_layer("d")(bd, mp2d, src_dp_d, dp0)
        rawp = _sc_layer("p")(bp, md2p, mp2p, src_dp_p, dp1, src_pp, pp1)
        hd2, hp2 = _act(rawd, hd2), _act(rawp, hp2)
    bd, mp2d = _dense2(
        hd2[:N_DRUGS], hd2[N_DRUGS:], hp2[:N_PROTS], hp2[N_PROTS:],
        W_res[nres - 1, 0], W_res[nres - 1, 1], b_res[nres - 1, 0])
    rawd = _sc_layer("d")(bd, mp2d, src_dp_d, dp0)
    hd2 = _act(rawd, hd2)

    pi = dd_pair_index[0].astype(i32)
    pj = dd_pair_index[1].astype(i32)
    idx_all = jnp.concatenate([pi, pi + N_DRUGS, pj, pj + N_DRUGS])
    pairs = _pair_gather(hd2, idx_all)
    comb = _mlp(pairs, Wp1, bp1, Wp2, bp2, Wp3, bp3)
    return comb[:, :, None]
